# baseline (speedup 1.0000x reference)
import functools

import jax
import jax.numpy as jnp
from jax.experimental import pallas as pl
from jax.experimental.pallas import tpu as pltpu

_B = 16


def _fused_mlp_segmax(x_ref, pos_ref, bat_ref, w1_ref, w2_ref, bias_ref,
                      out_ref, *, tile: int):
    j = pl.program_id(1)

    @pl.when(j == 0)
    def _init():
        out_ref[:] = jnp.full_like(out_ref, -jnp.inf)

    h = jnp.dot(x_ref[:], w1_ref[:], preferred_element_type=jnp.float32)
    h = h + jnp.dot(pos_ref[:], w2_ref[:], preferred_element_type=jnp.float32)
    h = jnp.maximum(h + bias_ref[:], 0.0)

    bt = bat_ref[:]
    first = bt[0, 0, 0]
    last = bt[0, 0, tile - 1]
    row = jax.lax.broadcasted_iota(jnp.int32, (tile, 1), 0)
    for s in range(_B):
        @pl.when(jnp.logical_and(first <= s, s <= last))
        def _seg(s=s):
            lo = jnp.sum((bt < s).astype(jnp.int32))
            hi = jnp.sum((bt <= s).astype(jnp.int32))
            m = jnp.logical_and(row >= lo, row < hi)
            seg = jnp.max(jnp.where(m, h, -jnp.inf), axis=0, keepdims=True)
            out_ref[0, s:s + 1, :] = jnp.maximum(out_ref[0, s:s + 1, :], seg)


def kernel(x, pos, batch, W, b):
    n, d = x.shape
    tile = 2000
    n_tiles = n // tile
    half = n_tiles // 2

    w1 = W[:d]
    w2 = W[d:]
    bias = b.reshape(1, d)
    bat3 = batch.astype(jnp.int32).reshape(n_tiles, 1, tile)

    parts = pl.pallas_call(
        functools.partial(_fused_mlp_segmax, tile=tile),
        grid=(2, half),
        in_specs=[
            pl.BlockSpec((tile, d), lambda i, j: (i * half + j, 0)),
            pl.BlockSpec((tile, 3), lambda i, j: (i * half + j, 0)),
            pl.BlockSpec((1, 1, tile), lambda i, j: (i * half + j, 0, 0)),
            pl.BlockSpec((d, d), lambda i, j: (0, 0)),
            pl.BlockSpec((3, d), lambda i, j: (0, 0)),
            pl.BlockSpec((1, d), lambda i, j: (0, 0)),
        ],
        out_specs=pl.BlockSpec((1, _B, d), lambda i, j: (i, 0, 0)),
        out_shape=jax.ShapeDtypeStruct((2, _B, d), jnp.float32),
        compiler_params=pltpu.CompilerParams(
            dimension_semantics=("parallel", "arbitrary")),
    )(x, pos, bat3, w1, w2, bias)

    pooled = jnp.maximum(parts[0], parts[1])
    pos_out = jnp.zeros((_B, 3), dtype=pos.dtype)
    batch_out = jnp.arange(_B, dtype=jnp.int64)
    return (pooled, pos_out, batch_out)


# speedup vs baseline: 1.2045x; 1.2045x over previous
import functools

import jax
import jax.numpy as jnp
from jax.experimental import pallas as pl
from jax.experimental.pallas import tpu as pltpu

_B = 16


def _fused_mlp_segmax(x_ref, pos_ref, bat_ref, w1_ref, w2_ref, bias_ref,
                      out_ref, *, tile: int):
    j = pl.program_id(0)

    @pl.when(j == 0)
    def _init():
        out_ref[:] = jnp.full_like(out_ref, -jnp.inf)

    h = jnp.dot(x_ref[:], w1_ref[:], preferred_element_type=jnp.float32)
    h = h + jnp.dot(pos_ref[:], w2_ref[:], preferred_element_type=jnp.float32)
    h = jnp.maximum(h + bias_ref[:], 0.0)

    bt = bat_ref[:]
    first = bt[0, 0, 0]
    last = bt[0, 0, tile - 1]
    row = jax.lax.broadcasted_iota(jnp.int32, (tile, 1), 0)
    for s in range(_B):
        @pl.when(jnp.logical_and(first <= s, s <= last))
        def _seg(s=s):
            lo = jnp.sum((bt < s).astype(jnp.int32))
            hi = jnp.sum((bt <= s).astype(jnp.int32))
            m = jnp.logical_and(row >= lo, row < hi)
            seg = jnp.max(jnp.where(m, h, -jnp.inf), axis=0, keepdims=True)
            out_ref[s:s + 1, :] = jnp.maximum(out_ref[s:s + 1, :], seg)


def kernel(x, pos, batch, W, b):
    n, d = x.shape
    tile = 4000
    n_tiles = n // tile

    w1 = W[:d]
    w2 = W[d:]
    bias = b.reshape(1, d)
    bat3 = batch.astype(jnp.int32).reshape(n_tiles, 1, tile)

    pooled = pl.pallas_call(
        functools.partial(_fused_mlp_segmax, tile=tile),
        grid=(n_tiles,),
        in_specs=[
            pl.BlockSpec((tile, d), lambda i: (i, 0)),
            pl.BlockSpec((tile, 3), lambda i: (i, 0)),
            pl.BlockSpec((1, 1, tile), lambda i: (i, 0, 0)),
            pl.BlockSpec((d, d), lambda i: (0, 0)),
            pl.BlockSpec((3, d), lambda i: (0, 0)),
            pl.BlockSpec((1, d), lambda i: (0, 0)),
        ],
        out_specs=pl.BlockSpec((_B, d), lambda i: (0, 0)),
        out_shape=jax.ShapeDtypeStruct((_B, d), jnp.float32),
    )(x, pos, bat3, w1, w2, bias)

    pos_out = jnp.zeros((_B, 3), dtype=pos.dtype)
    batch_out = jnp.arange(_B, dtype=jnp.int64)
    return (pooled, pos_out, batch_out)


# speedup vs baseline: 1.2055x; 1.0009x over previous
import functools

import jax
import jax.numpy as jnp
from jax.experimental import pallas as pl
from jax.experimental.pallas import tpu as pltpu

_B = 16


def _fused_mlp_segmax(x_ref, pos_ref, bat_ref, w1_ref, w2_ref, bias_ref,
                      out_ref, *, tile: int, n_tiles: int):
    i = pl.program_id(0)

    @pl.when(i == 0)
    def _init():
        out_ref[:] = jnp.full_like(out_ref, -jnp.inf)

    h = jnp.dot(x_ref[:], w1_ref[:], preferred_element_type=jnp.float32)
    h = h + jnp.dot(pos_ref[:], w2_ref[:], preferred_element_type=jnp.float32)

    bt = bat_ref[0]
    first = bt[0, 0]
    last = bt[7, tile // 8 - 1]

    @pl.when(first == last)
    def _single():
        seg = jnp.max(h, axis=0, keepdims=True)
        s1 = pl.ds(first, 1)
        out_ref[s1, :] = jnp.maximum(out_ref[s1, :], seg)

    @pl.when(first != last)
    def _multi():
        row = jax.lax.broadcasted_iota(jnp.int32, (tile, 1), 0)
        for s in range(_B):
            @pl.when(jnp.logical_and(first <= s, s <= last))
            def _seg(s=s):
                lo = jnp.sum((bt < s).astype(jnp.int32))
                hi = jnp.sum((bt <= s).astype(jnp.int32))
                m = jnp.logical_and(row >= lo, row < hi)
                seg = jnp.max(jnp.where(m, h, -jnp.inf), axis=0, keepdims=True)
                out_ref[s:s + 1, :] = jnp.maximum(out_ref[s:s + 1, :], seg)

    @pl.when(i == n_tiles - 1)
    def _fixup():
        acc = out_ref[:]
        res = jnp.maximum(acc + bias_ref[:], 0.0)
        out_ref[:] = jnp.where(acc == -jnp.inf, acc, res)


def kernel(x, pos, batch, W, b):
    n, d = x.shape
    tile = 4000
    n_tiles = n // tile

    w1 = W[:d]
    w2 = W[d:]
    bias = b.reshape(1, d)
    bat3 = batch.astype(jnp.int32).reshape(n_tiles, 8, tile // 8)

    pooled = pl.pallas_call(
        functools.partial(_fused_mlp_segmax, tile=tile, n_tiles=n_tiles),
        grid=(n_tiles,),
        in_specs=[
            pl.BlockSpec((tile, d), lambda i: (i, 0)),
            pl.BlockSpec((tile, 3), lambda i: (i, 0)),
            pl.BlockSpec((1, 8, tile // 8), lambda i: (i, 0, 0)),
            pl.BlockSpec((d, d), lambda i: (0, 0)),
            pl.BlockSpec((3, d), lambda i: (0, 0)),
            pl.BlockSpec((1, d), lambda i: (0, 0)),
        ],
        out_specs=pl.BlockSpec((_B, d), lambda i: (0, 0)),
        out_shape=jax.ShapeDtypeStruct((_B, d), jnp.float32),
    )(x, pos, bat3, w1, w2, bias)

    pos_out = jnp.zeros((_B, 3), dtype=pos.dtype)
    batch_out = jnp.arange(_B, dtype=jnp.int64)
    return (pooled, pos_out, batch_out)


# speedup vs baseline: 1.7009x; 1.4109x over previous
import functools

import jax
import jax.numpy as jnp
from jax.experimental import pallas as pl
from jax.experimental.pallas import tpu as pltpu

_B = 16


def _fused_mlp_segmax(x_ref, posT_ref, bat_ref, w1_ref, w2_ref,
                      out_ref, *, tile: int, n_tiles: int):
    i = pl.program_id(0)

    @pl.when(i == 0)
    def _init():
        out_ref[:] = jnp.full_like(out_ref, -jnp.inf)

    h = jnp.dot(x_ref[:], w1_ref[:], preferred_element_type=jnp.float32)
    h = h + jax.lax.dot_general(
        posT_ref[:], w2_ref[:],
        dimension_numbers=(((0,), (0,)), ((), ())),
        preferred_element_type=jnp.float32)

    bt = bat_ref[0]
    first = bt[0, 0]
    last = bt[7, tile // 8 - 1]

    @pl.when(first == last)
    def _single():
        seg = jnp.max(h, axis=0, keepdims=True)
        s1 = pl.ds(first, 1)
        out_ref[s1, :] = jnp.maximum(out_ref[s1, :], seg)

    @pl.when(first != last)
    def _multi():
        row = jax.lax.broadcasted_iota(jnp.int32, (tile, 1), 0)
        for s in range(_B):
            @pl.when(jnp.logical_and(first <= s, s <= last))
            def _seg(s=s):
                lo = jnp.sum((bt < s).astype(jnp.int32))
                hi = jnp.sum((bt <= s).astype(jnp.int32))
                m = jnp.logical_and(row >= lo, row < hi)
                seg = jnp.max(jnp.where(m, h, -jnp.inf), axis=0, keepdims=True)
                out_ref[s:s + 1, :] = jnp.maximum(out_ref[s:s + 1, :], seg)

    @pl.when(i == n_tiles - 1)
    def _fixup():
        acc = out_ref[:]
        out_ref[:] = jnp.where(acc == -jnp.inf, acc, jnp.maximum(acc, 0.0))


def kernel(x, pos, batch, W, b):
    n, d = x.shape
    tile = 4096
    n_tiles = (n + tile - 1) // tile
    n_pad = n_tiles * tile - n

    w1 = W[:d]
    w2 = jnp.concatenate([W[d:], b.reshape(1, d)])
    posT = jnp.pad(
        jnp.concatenate([pos.T, jnp.ones((1, n), pos.dtype)], axis=0),
        ((0, 0), (0, n_pad)))
    bat3 = jnp.pad(batch.astype(jnp.int32), (0, n_pad),
                   constant_values=_B).reshape(n_tiles, 8, tile // 8)

    pooled = pl.pallas_call(
        functools.partial(_fused_mlp_segmax, tile=tile, n_tiles=n_tiles),
        grid=(n_tiles,),
        in_specs=[
            pl.BlockSpec((tile, d), lambda i: (i, 0)),
            pl.BlockSpec((4, tile), lambda i: (0, i)),
            pl.BlockSpec((1, 8, tile // 8), lambda i: (i, 0, 0)),
            pl.BlockSpec((d, d), lambda i: (0, 0)),
            pl.BlockSpec((4, d), lambda i: (0, 0)),
        ],
        out_specs=pl.BlockSpec((_B, d), lambda i: (0, 0)),
        out_shape=jax.ShapeDtypeStruct((_B, d), jnp.float32),
    )(x, posT, bat3, w1, w2)

    pos_out = jnp.zeros((_B, 3), dtype=pos.dtype)
    batch_out = jnp.arange(_B, dtype=jnp.int64)
    return (pooled, pos_out, batch_out)
